# baseline (device time: 47872 ns/iter reference)
import jax
import jax.numpy as jnp
from jax import lax
from jax.experimental import pallas as pl
from jax.experimental.pallas import tpu as pltpu

N_DEV = 4
B, SQ, SKV = 2, 512, 512
HQ_LOC, DH = 8, 64
DM = 768
HALF = DM // 2
DQ_LOC = HQ_LOC * DH
ROWS = B * SQ
CHUNK = ROWS // N_DEV
NH = N_DEV - 1


def kernel(x, Wq, K_ext, V_ext, Wo):
    i = lax.axis_index("i")
    Wq_loc = (lax.dynamic_slice(Wq, (0, i * DQ_LOC), (DM, DQ_LOC)) * 0.125
              ).astype(jnp.bfloat16)
    Wo_loc = lax.dynamic_slice(Wo, (i * DQ_LOC, 0), (DQ_LOC, DM)
                               ).astype(jnp.bfloat16)
    x16 = x.astype(jnp.bfloat16)
    K16 = K_ext.astype(jnp.bfloat16)
    V16 = V_ext.astype(jnp.bfloat16)

    def body(x_ref, wq_ref, k_ref, v_ref, wo_ref, out_ref,
             acc_ref, q_ref, bias_ref, ctx_ref,
             rs_a, rs_b, snd_a, snd_b, ag_ref,
             send_sems, recv_sems):
        my = lax.axis_index("i")
        left = lax.rem(my + N_DEV - 1, N_DEV)
        right = lax.rem(my + 1, N_DEV)

        barrier_sem = pltpu.get_barrier_semaphore()
        for nbr in (left, right):
            pl.semaphore_signal(
                barrier_sem, inc=1,
                device_id=(nbr,), device_id_type=pl.DeviceIdType.MESH,
            )
        pl.semaphore_wait(barrier_sem, 2)

        qi = lax.broadcasted_iota(jnp.int32, (SQ, SKV), 0)
        ki = lax.broadcasted_iota(jnp.int32, (SQ, SKV), 1)
        d = qi - ki
        mask = ((d <= 128) & (d >= -128)) | (ki < 32) | (qi < 32)
        bias_ref[:, :] = jnp.where(mask, 0.0, -1e9).astype(jnp.float32)

        for b in range(B):
            q_ref[pl.ds(b * SQ, SQ), :] = jnp.dot(
                x_ref[b, :, :], wq_ref[:, :],
                preferred_element_type=jnp.float32).astype(jnp.bfloat16)

        def compute_chunk(c):
            r0 = c * CHUNK
            b = c // 2
            rq0 = (c % 2) * CHUNK
            biasc = bias_ref[pl.ds(rq0, CHUNK), :]
            for h in range(HQ_LOC):
                qh = q_ref[pl.ds(r0, CHUNK), h * DH:(h + 1) * DH]
                kh = k_ref[b, :, h, :]
                vh = v_ref[b, :, h, :]
                s = lax.dot_general(
                    qh, kh, (((1,), (1,)), ((), ())),
                    preferred_element_type=jnp.float32)
                w = jnp.exp(s + biasc)
                denom = jnp.sum(w, axis=-1, keepdims=True)
                ctx = jnp.dot(w.astype(jnp.bfloat16), vh,
                              preferred_element_type=jnp.float32)
                ctx_ref[:, h * DH:(h + 1) * DH] = (
                    ctx / denom).astype(jnp.bfloat16)
            acc_ref[pl.ds(r0, CHUNK), :] = jnp.dot(
                ctx_ref[:, :], wo_ref[:, :],
                preferred_element_type=jnp.float32)

        def start_rs(s):
            csa = lax.rem(my - s + N_DEV, N_DEV)
            csb = lax.rem(my + s, N_DEV)
            snd_a[:, :] = acc_ref[pl.ds(csa * CHUNK, CHUNK),
                                  :HALF].astype(jnp.bfloat16)
            snd_b[:, :] = acc_ref[pl.ds(csb * CHUNK, CHUNK),
                                  HALF:].astype(jnp.bfloat16)
            ra = pltpu.make_async_remote_copy(
                src_ref=snd_a, dst_ref=rs_a.at[s],
                send_sem=send_sems.at[2 * s], recv_sem=recv_sems.at[2 * s],
                device_id=(right,), device_id_type=pl.DeviceIdType.MESH,
            )
            rb = pltpu.make_async_remote_copy(
                src_ref=snd_b, dst_ref=rs_b.at[s],
                send_sem=send_sems.at[2 * s + 1],
                recv_sem=recv_sems.at[2 * s + 1],
                device_id=(left,), device_id_type=pl.DeviceIdType.MESH,
            )
            ra.start()
            rb.start()
            return ra, rb

        def finish_rs(s, ra, rb):
            cra = lax.rem(my - s - 1 + N_DEV, N_DEV)
            crb = lax.rem(my + s + 1, N_DEV)
            ra.wait()
            rb.wait()
            acc_ref[pl.ds(cra * CHUNK, CHUNK), :HALF] = (
                acc_ref[pl.ds(cra * CHUNK, CHUNK), :HALF]
                + rs_a[s, :, :].astype(jnp.float32)
            )
            acc_ref[pl.ds(crb * CHUNK, CHUNK), HALF:] = (
                acc_ref[pl.ds(crb * CHUNK, CHUNK), HALF:]
                + rs_b[s, :, :].astype(jnp.float32)
            )

        compute_chunk(my)
        h0 = start_rs(0)
        compute_chunk(lax.rem(my + 1, N_DEV))
        compute_chunk(lax.rem(my + 3, N_DEV))
        finish_rs(0, *h0)
        h1 = start_rs(1)
        compute_chunk(lax.rem(my + 2, N_DEV))
        finish_rs(1, *h1)
        h2 = start_rs(2)
        finish_rs(2, *h2)

        owna = lax.rem(my + 1, N_DEV)
        ownb = lax.rem(my + N_DEV - 1, N_DEV)
        ag_ref[pl.ds(owna * CHUNK, CHUNK), :HALF] = acc_ref[
            pl.ds(owna * CHUNK, CHUNK), :HALF].astype(jnp.bfloat16)
        ag_ref[pl.ds(ownb * CHUNK, CHUNK), HALF:] = acc_ref[
            pl.ds(ownb * CHUNK, CHUNK), HALF:].astype(jnp.bfloat16)

        for t in range(NH):
            ca = lax.rem(my + 1 - t + N_DEV, N_DEV)
            cb = lax.rem(my - 1 + t + N_DEV, N_DEV)
            ra = pltpu.make_async_remote_copy(
                src_ref=ag_ref.at[pl.ds(ca * CHUNK, CHUNK), pl.ds(0, HALF)],
                dst_ref=ag_ref.at[pl.ds(ca * CHUNK, CHUNK), pl.ds(0, HALF)],
                send_sem=send_sems.at[2 * NH + 2 * t],
                recv_sem=recv_sems.at[2 * NH + 2 * t],
                device_id=(right,), device_id_type=pl.DeviceIdType.MESH,
            )
            rb = pltpu.make_async_remote_copy(
                src_ref=ag_ref.at[pl.ds(cb * CHUNK, CHUNK), pl.ds(HALF, HALF)],
                dst_ref=ag_ref.at[pl.ds(cb * CHUNK, CHUNK), pl.ds(HALF, HALF)],
                send_sem=send_sems.at[2 * NH + 2 * t + 1],
                recv_sem=recv_sems.at[2 * NH + 2 * t + 1],
                device_id=(left,), device_id_type=pl.DeviceIdType.MESH,
            )
            ra.start()
            rb.start()
            ra.wait()
            rb.wait()

        out_ref[0, :, :] = ag_ref[pl.ds(0, SQ), :].astype(jnp.float32)
        out_ref[1, :, :] = ag_ref[pl.ds(SQ, SQ), :].astype(jnp.float32)

    return pl.pallas_call(
        body,
        out_shape=jax.ShapeDtypeStruct((B, SQ, DM), jnp.float32),
        in_specs=[pl.BlockSpec(memory_space=pltpu.VMEM)] * 5,
        out_specs=pl.BlockSpec(memory_space=pltpu.VMEM),
        scratch_shapes=[
            pltpu.VMEM((ROWS, DM), jnp.float32),
            pltpu.VMEM((ROWS, DQ_LOC), jnp.bfloat16),
            pltpu.VMEM((SQ, SKV), jnp.float32),
            pltpu.VMEM((CHUNK, DQ_LOC), jnp.bfloat16),
            pltpu.VMEM((NH, CHUNK, HALF), jnp.bfloat16),
            pltpu.VMEM((NH, CHUNK, HALF), jnp.bfloat16),
            pltpu.VMEM((CHUNK, HALF), jnp.bfloat16),
            pltpu.VMEM((CHUNK, HALF), jnp.bfloat16),
            pltpu.VMEM((ROWS, DM), jnp.bfloat16),
            pltpu.SemaphoreType.DMA((4 * NH,)),
            pltpu.SemaphoreType.DMA((4 * NH,)),
        ],
        compiler_params=pltpu.CompilerParams(collective_id=0),
    )(x16, Wq_loc, K16, V16, Wo_loc)


# device time: 46468 ns/iter; 1.0302x vs baseline; 1.0302x over previous
import jax
import jax.numpy as jnp
from jax import lax
from jax.experimental import pallas as pl
from jax.experimental.pallas import tpu as pltpu

N_DEV = 4
B, SQ, SKV = 2, 512, 512
HQ_LOC, DH = 8, 64
DM = 768
DQ_LOC = HQ_LOC * DH
ROWS = B * SQ
CHUNK = ROWS // N_DEV


def kernel(x, Wq, K_ext, V_ext, Wo):
    i = lax.axis_index("i")
    Wq_loc = (lax.dynamic_slice(Wq, (0, i * DQ_LOC), (DM, DQ_LOC)) * 0.125
              ).astype(jnp.bfloat16)
    Wo_loc = lax.dynamic_slice(Wo, (i * DQ_LOC, 0), (DQ_LOC, DM)
                               ).astype(jnp.bfloat16)
    x16 = x.astype(jnp.bfloat16)
    K16 = K_ext.astype(jnp.bfloat16)
    V16 = V_ext.astype(jnp.bfloat16)

    def body(x_ref, wq_ref, k_ref, v_ref, wo_ref, out_ref,
             acc_ref, ctx_ref, snd_rs, rs_buf, ag_ref,
             send_sems, recv_sems):
        my = lax.axis_index("i")

        barrier_sem = pltpu.get_barrier_semaphore()
        for d in range(1, N_DEV):
            pl.semaphore_signal(
                barrier_sem, inc=1,
                device_id=(lax.rem(my + d, N_DEV),),
                device_id_type=pl.DeviceIdType.MESH,
            )
        pl.semaphore_wait(barrier_sem, N_DEV - 1)

        qi = lax.broadcasted_iota(jnp.int32, (SQ, SKV), 0)
        ki = lax.broadcasted_iota(jnp.int32, (SQ, SKV), 1)
        dd = qi - ki
        mask = ((dd <= 128) & (dd >= -128)) | (ki < 32) | (qi < 32)
        bias = jnp.where(mask, 0.0, -1e9).astype(jnp.float32)

        def rs_send(d):
            c = lax.rem(my + d, N_DEV)
            return pltpu.make_async_remote_copy(
                src_ref=snd_rs.at[d - 1],
                dst_ref=rs_buf.at[d - 1],
                send_sem=send_sems.at[d - 1],
                recv_sem=recv_sems.at[d - 1],
                device_id=(c,),
                device_id_type=pl.DeviceIdType.MESH,
            )

        def rs_stage_and_send(d):
            c = lax.rem(my + d, N_DEV)
            snd_rs[d - 1, :, :] = acc_ref[pl.ds(c * CHUNK, CHUNK),
                                          :].astype(jnp.bfloat16)
            rs_send(d).start()

        for b in range(B):
            xb = x_ref[b, :, :]
            q = jnp.dot(xb, wq_ref[:, :],
                        preferred_element_type=jnp.float32
                        ).astype(jnp.bfloat16)
            for h in range(HQ_LOC):
                qh = q[:, h * DH:(h + 1) * DH]
                kh = k_ref[b, :, h, :]
                vh = v_ref[b, :, h, :]
                s = lax.dot_general(
                    qh, kh, (((1,), (1,)), ((), ())),
                    preferred_element_type=jnp.float32)
                w = jnp.exp(s + bias)
                denom = jnp.sum(w, axis=-1, keepdims=True)
                ctx = jnp.dot(w.astype(jnp.bfloat16), vh,
                              preferred_element_type=jnp.float32)
                ctx_ref[:, h * DH:(h + 1) * DH] = (
                    ctx / denom).astype(jnp.bfloat16)
            acc_ref[pl.ds(b * SQ, SQ), :] = jnp.dot(
                ctx_ref[:, :], wo_ref[:, :],
                preferred_element_type=jnp.float32)

            for d in range(1, N_DEV):
                c = lax.rem(my + d, N_DEV)
                if b == 0:
                    @pl.when(c < 2)
                    def _(d=d):
                        rs_stage_and_send(d)
                else:
                    @pl.when(c >= 2)
                    def _(d=d):
                        rs_stage_and_send(d)

        for d in range(1, N_DEV):
            rs_send(d).wait_recv()
        red = acc_ref[pl.ds(my * CHUNK, CHUNK), :]
        for d in range(1, N_DEV):
            red = red + rs_buf[d - 1, :, :].astype(jnp.float32)
        ag_ref[pl.ds(my * CHUNK, CHUNK), :] = red.astype(jnp.bfloat16)

        ag_rdmas = []
        for d in range(1, N_DEV):
            r = pltpu.make_async_remote_copy(
                src_ref=ag_ref.at[pl.ds(my * CHUNK, CHUNK), :],
                dst_ref=ag_ref.at[pl.ds(my * CHUNK, CHUNK), :],
                send_sem=send_sems.at[N_DEV - 1 + d - 1],
                recv_sem=recv_sems.at[N_DEV - 1 + d - 1],
                device_id=(lax.rem(my + d, N_DEV),),
                device_id_type=pl.DeviceIdType.MESH,
            )
            r.start()
            ag_rdmas.append(r)

        for d in range(1, N_DEV):
            src = lax.rem(my - d + N_DEV, N_DEV)
            pltpu.make_async_remote_copy(
                src_ref=ag_ref.at[pl.ds(src * CHUNK, CHUNK), :],
                dst_ref=ag_ref.at[pl.ds(src * CHUNK, CHUNK), :],
                send_sem=send_sems.at[N_DEV - 1 + d - 1],
                recv_sem=recv_sems.at[N_DEV - 1 + d - 1],
                device_id=(src,),
                device_id_type=pl.DeviceIdType.MESH,
            ).wait_recv()

        for d in range(1, N_DEV):
            rs_send(d).wait_send()
        for r in ag_rdmas:
            r.wait_send()

        out_ref[0, :, :] = ag_ref[pl.ds(0, SQ), :].astype(jnp.float32)
        out_ref[1, :, :] = ag_ref[pl.ds(SQ, SQ), :].astype(jnp.float32)

    return pl.pallas_call(
        body,
        out_shape=jax.ShapeDtypeStruct((B, SQ, DM), jnp.float32),
        in_specs=[pl.BlockSpec(memory_space=pltpu.VMEM)] * 5,
        out_specs=pl.BlockSpec(memory_space=pltpu.VMEM),
        scratch_shapes=[
            pltpu.VMEM((ROWS, DM), jnp.float32),
            pltpu.VMEM((SQ, DQ_LOC), jnp.bfloat16),
            pltpu.VMEM((N_DEV - 1, CHUNK, DM), jnp.bfloat16),
            pltpu.VMEM((N_DEV - 1, CHUNK, DM), jnp.bfloat16),
            pltpu.VMEM((ROWS, DM), jnp.bfloat16),
            pltpu.SemaphoreType.DMA((2 * (N_DEV - 1),)),
            pltpu.SemaphoreType.DMA((2 * (N_DEV - 1),)),
        ],
        compiler_params=pltpu.CompilerParams(collective_id=0),
    )(x16, Wq_loc, K16, V16, Wo_loc)
